# baseline (speedup 1.0000x reference)
import functools

import jax
import jax.numpy as jnp
from jax import lax
from jax.experimental import pallas as pl
from jax.experimental.pallas import tpu as pltpu
from jax.experimental.pallas import tpu_sc as plsc

T = 8192
D = 2048
E = 64
TOPK = 2
TILE_T = 1024

NW = 32
TPW = T // NW
LANES = 16
GROUPS = TPW // LANES


def _scores_kernel(x_ref, wg_ref, wu_ref, sm_out_ref, wfull_ref):
    @pl.when(pl.program_id(0) == 0)
    def _init():
        wfull_ref[:, :E] = wg_ref[...]
        wfull_ref[:, E:] = wu_ref[...]

    acc = jax.lax.dot_general(
        wfull_ref[...], x_ref[...],
        dimension_numbers=(((0,), (1,)), ((), ())),
        preferred_element_type=jnp.float32,
    )
    gate = acc[:E, :]
    up = acc[E:, :]
    s = jnp.abs(up * gate * jax.nn.sigmoid(gate))
    ex = jnp.exp(jnp.minimum(s, 80.0))
    sm_out_ref[...] = ex / jnp.sum(ex, axis=0, keepdims=True)


def _tc_scores(x, W_gate, W_up):
    return pl.pallas_call(
        _scores_kernel,
        grid=(T // TILE_T,),
        in_specs=[
            pl.BlockSpec((TILE_T, D), lambda i: (i, 0)),
            pl.BlockSpec((D, E), lambda i: (0, 0)),
            pl.BlockSpec((D, E), lambda i: (0, 0)),
        ],
        out_specs=pl.BlockSpec((E, TILE_T), lambda i: (0, i)),
        out_shape=jax.ShapeDtypeStruct((E, T), jnp.float32),
        scratch_shapes=[pltpu.VMEM((D, 2 * E), jnp.float32)],
    )(x, W_gate, W_up)


def _sc_router(sm_hbm, bias_hbm, scale_hbm, w_hbm, i_hbm,
               sm_v, bias_v, scale_v, w_v, i_v):
    wid = lax.axis_index("s") * 2 + lax.axis_index("c")
    base = wid * TPW
    pltpu.sync_copy(sm_hbm.at[:, pl.ds(base, TPW)], sm_v)
    pltpu.sync_copy(bias_hbm, bias_v)
    pltpu.sync_copy(scale_hbm, scale_v)

    neg = jnp.full((LANES,), -jnp.inf, jnp.float32)

    def group(k, carry):
        off = k * LANES
        entries = []
        for e in range(E):
            sm = sm_v[e, pl.ds(off, LANES)]
            val = sm + bias_v[e, :]
            w = 1.0 + sm * scale_v[e, :]
            idx = jnp.full((LANES,), e, jnp.int32)
            entries.append((val, idx, w))

        def tree(items):
            while len(items) > 1:
                nxt = []
                for j in range(0, len(items) - 1, 2):
                    av, ai, aw = items[j]
                    bv, bi, bw = items[j + 1]
                    p = av >= bv
                    nxt.append((jnp.where(p, av, bv),
                                jnp.where(p, ai, bi),
                                jnp.where(p, aw, bw)))
                if len(items) % 2:
                    nxt.append(items[-1])
                items = nxt
            return items[0]

        m1, i1, w1 = tree(entries)
        masked = []
        for e in range(E):
            val, idx, w = entries[e]
            val = jnp.where(i1 == e, neg, val)
            masked.append((val, idx, w))
        m2, i2, w2 = tree(masked)

        w_v[0, pl.ds(off, LANES)] = w1
        w_v[1, pl.ds(off, LANES)] = w2
        i_v[0, pl.ds(off, LANES)] = i1
        i_v[1, pl.ds(off, LANES)] = i2
        return carry

    lax.fori_loop(0, GROUPS, group, 0)

    pltpu.sync_copy(w_v, w_hbm.at[:, pl.ds(base, TPW)])
    pltpu.sync_copy(i_v, i_hbm.at[:, pl.ds(base, TPW)])


_sc_call = pl.kernel(
    _sc_router,
    out_type=[
        jax.ShapeDtypeStruct((TOPK, T), jnp.float32),
        jax.ShapeDtypeStruct((TOPK, T), jnp.int32),
    ],
    mesh=plsc.VectorSubcoreMesh(core_axis_name="c", subcore_axis_name="s"),
    scratch_types=[
        pltpu.VMEM((E, TPW), jnp.float32),
        pltpu.VMEM((E, LANES), jnp.float32),
        pltpu.VMEM((E, LANES), jnp.float32),
        pltpu.VMEM((TOPK, TPW), jnp.float32),
        pltpu.VMEM((TOPK, TPW), jnp.int32),
    ],
)


@jax.jit
def kernel(x, W_gate, W_up, extra_scale, extra_bias):
    sm_t = _tc_scores(x, W_gate, W_up)
    bias_b = jnp.broadcast_to(extra_bias.reshape(E, 1), (E, LANES))
    scale_b = jnp.broadcast_to(extra_scale.reshape(E, 1), (E, LANES))
    w_t, i_t = _sc_call(sm_t, bias_b, scale_b)
    return w_t.T, i_t.T


# speedup vs baseline: 1.7925x; 1.7925x over previous
import jax
import jax.numpy as jnp
from jax.experimental import pallas as pl
from jax.experimental.pallas import tpu as pltpu

T = 8192
D = 2048
E = 64
TOPK = 2
TILE_T = 1024


def _router_kernel(xa_ref, xb_ref, wg_ref, wu_ref, bias_ref, scale_ref,
                   w_out_ref, i_out_ref, wfull_ref):
    @pl.when(pl.program_id(0) == 0)
    def _init():
        wfull_ref[:, :E] = wg_ref[...]
        wfull_ref[:, E:] = wu_ref[...]

    dn = (((0,), (1,)), ((), ()))
    acc = jax.lax.dot_general(
        wfull_ref[: D // 2, :], xa_ref[...],
        dimension_numbers=dn, preferred_element_type=jnp.float32,
    ) + jax.lax.dot_general(
        wfull_ref[D // 2 :, :], xb_ref[...],
        dimension_numbers=dn, preferred_element_type=jnp.float32,
    )
    gate = acc[:E, :]
    up = acc[E:, :]
    s = jnp.abs(up * gate * jax.nn.sigmoid(gate))
    ex = jnp.exp(jnp.minimum(s, 80.0))
    sm = ex / jnp.sum(ex, axis=0, keepdims=True)

    biased = sm + bias_ref[...]
    row = jax.lax.broadcasted_iota(jnp.int32, biased.shape, 0)

    m1 = jnp.max(biased, axis=0, keepdims=True)
    i1 = jnp.min(jnp.where(biased == m1, row, E), axis=0, keepdims=True)
    mask1 = row == i1
    rest = jnp.where(mask1, -jnp.inf, biased)
    m2 = jnp.max(rest, axis=0, keepdims=True)
    i2 = jnp.min(jnp.where(rest == m2, row, E), axis=0, keepdims=True)
    mask2 = row == i2

    w = 1.0 + sm * scale_ref[...]
    w1 = jnp.sum(jnp.where(mask1, w, 0.0), axis=0, keepdims=True)
    w2 = jnp.sum(jnp.where(mask2, w, 0.0), axis=0, keepdims=True)

    w_out_ref[...] = jnp.concatenate([w1, w2], axis=0)
    i_out_ref[...] = jnp.concatenate([i1, i2], axis=0)


@jax.jit
def kernel(x, W_gate, W_up, extra_scale, extra_bias):
    bias2d = extra_bias.reshape(E, 1)
    scale2d = extra_scale.reshape(E, 1)
    grid = (T // TILE_T,)
    weights, indices = pl.pallas_call(
        _router_kernel,
        grid=grid,
        in_specs=[
            pl.BlockSpec((TILE_T, D // 2), lambda i: (i, 0)),
            pl.BlockSpec((TILE_T, D // 2), lambda i: (i, 1)),
            pl.BlockSpec((D, E), lambda i: (0, 0)),
            pl.BlockSpec((D, E), lambda i: (0, 0)),
            pl.BlockSpec((E, 1), lambda i: (0, 0)),
            pl.BlockSpec((E, 1), lambda i: (0, 0)),
        ],
        out_specs=[
            pl.BlockSpec((TOPK, TILE_T), lambda i: (0, i)),
            pl.BlockSpec((TOPK, TILE_T), lambda i: (0, i)),
        ],
        out_shape=[
            jax.ShapeDtypeStruct((TOPK, T), jnp.float32),
            jax.ShapeDtypeStruct((TOPK, T), jnp.int32),
        ],
        scratch_shapes=[pltpu.VMEM((D, 2 * E), jnp.float32)],
    )(x, x, W_gate, W_up, bias2d, scale2d)
    return weights.T, indices.T


# speedup vs baseline: 1.7950x; 1.0014x over previous
import jax
import jax.numpy as jnp
from jax.experimental import pallas as pl
from jax.experimental.pallas import tpu as pltpu

T = 8192
D = 2048
E = 64
TOPK = 2
TILE_T = 1024


def _router_kernel(x_ref, wg_ref, wu_ref, bias_ref, scale_ref,
                   w_out_ref, i_out_ref, wfull_ref):
    @pl.when(pl.program_id(0) == 0)
    def _init():
        wfull_ref[:, :E] = wg_ref[...]
        wfull_ref[:, E:] = wu_ref[...]

    acc = jax.lax.dot_general(
        wfull_ref[...], x_ref[...],
        dimension_numbers=(((0,), (1,)), ((), ())),
        preferred_element_type=jnp.float32,
    )
    gate = acc[:E, :]
    up = acc[E:, :]
    s = jnp.abs(up * gate * jax.nn.sigmoid(gate))
    ex = jnp.exp(jnp.minimum(s, 80.0))
    sm = ex / jnp.sum(ex, axis=0, keepdims=True)

    biased = sm + bias_ref[...]
    row = jax.lax.broadcasted_iota(jnp.int32, biased.shape, 0)

    m1 = jnp.max(biased, axis=0, keepdims=True)
    i1 = jnp.min(jnp.where(biased == m1, row, E), axis=0, keepdims=True)
    mask1 = row == i1
    rest = jnp.where(mask1, -jnp.inf, biased)
    m2 = jnp.max(rest, axis=0, keepdims=True)
    i2 = jnp.min(jnp.where(rest == m2, row, E), axis=0, keepdims=True)
    mask2 = row == i2

    w = 1.0 + sm * scale_ref[...]
    w1 = jnp.sum(jnp.where(mask1, w, 0.0), axis=0, keepdims=True)
    w2 = jnp.sum(jnp.where(mask2, w, 0.0), axis=0, keepdims=True)

    w_out_ref[...] = jnp.concatenate([w1, w2], axis=0)
    i_out_ref[...] = jnp.concatenate([i1, i2], axis=0)


@jax.jit
def kernel(x, W_gate, W_up, extra_scale, extra_bias):
    bias2d = extra_bias.reshape(E, 1)
    scale2d = extra_scale.reshape(E, 1)
    grid = (T // TILE_T,)
    weights, indices = pl.pallas_call(
        _router_kernel,
        grid=grid,
        in_specs=[
            pl.BlockSpec((TILE_T, D), lambda i: (i, 0)),
            pl.BlockSpec((D, E), lambda i: (0, 0)),
            pl.BlockSpec((D, E), lambda i: (0, 0)),
            pl.BlockSpec((E, 1), lambda i: (0, 0)),
            pl.BlockSpec((E, 1), lambda i: (0, 0)),
        ],
        out_specs=[
            pl.BlockSpec((TOPK, TILE_T), lambda i: (0, i)),
            pl.BlockSpec((TOPK, TILE_T), lambda i: (0, i)),
        ],
        out_shape=[
            jax.ShapeDtypeStruct((TOPK, T), jnp.float32),
            jax.ShapeDtypeStruct((TOPK, T), jnp.int32),
        ],
        scratch_shapes=[pltpu.VMEM((D, 2 * E), jnp.float32)],
    )(x, W_gate, W_up, bias2d, scale2d)
    return weights.T, indices.T


# speedup vs baseline: 1.8321x; 1.0207x over previous
import jax
import jax.numpy as jnp
from jax import lax
from jax.experimental import pallas as pl
from jax.experimental.pallas import tpu as pltpu

T = 8192
D = 2048
E = 64
TOPK = 2
TILE_T = 1024
NT = T // TILE_T


def _routing_tail(acc, bias, scale, w_out_ref, i_out_ref, off):
    gate = acc[:E, :]
    up = acc[E:, :]
    s = jnp.abs(up * gate * jax.nn.sigmoid(gate))
    ex = jnp.exp(jnp.minimum(s, 80.0))
    sm = ex / jnp.sum(ex, axis=0, keepdims=True)

    biased = sm + bias
    row = jax.lax.broadcasted_iota(jnp.int32, biased.shape, 0)

    m1 = jnp.max(biased, axis=0, keepdims=True)
    i1 = jnp.min(jnp.where(biased == m1, row, E), axis=0, keepdims=True)
    mask1 = row == i1
    rest = jnp.where(mask1, -jnp.inf, biased)
    m2 = jnp.max(rest, axis=0, keepdims=True)
    i2 = jnp.min(jnp.where(rest == m2, row, E), axis=0, keepdims=True)
    mask2 = row == i2

    w = 1.0 + sm * scale
    w1 = jnp.sum(jnp.where(mask1, w, 0.0), axis=0, keepdims=True)
    w2 = jnp.sum(jnp.where(mask2, w, 0.0), axis=0, keepdims=True)

    w_out_ref[:, pl.ds(off, TILE_T)] = jnp.concatenate([w1, w2], axis=0)
    i_out_ref[:, pl.ds(off, TILE_T)] = jnp.concatenate([i1, i2], axis=0)


def _router_kernel(x_hbm, wg_ref, wu_ref, bias_ref, scale_ref,
                   w_out_ref, i_out_ref, xbuf_ref, wfull_ref, sems):
    wfull_ref[:, :E] = wg_ref[...]
    wfull_ref[:, E:] = wu_ref[...]
    bias = bias_ref[...]
    scale = scale_ref[...]
    dn = (((0,), (1,)), ((), ()))

    def copy(i, slot):
        return pltpu.make_async_copy(
            x_hbm.at[pl.ds(i * TILE_T, TILE_T), :],
            xbuf_ref.at[slot],
            sems.at[slot],
        )

    copy(0, 0).start()
    copy(1, 1).start()

    def step(k, carry):
        i0 = 2 * k
        copy(i0, 0).wait()
        acc = jax.lax.dot_general(
            wfull_ref[...], xbuf_ref[0],
            dimension_numbers=dn, preferred_element_type=jnp.float32)
        _routing_tail(acc, bias, scale, w_out_ref, i_out_ref, i0 * TILE_T)

        @pl.when(i0 + 2 < NT)
        def _pf0():
            copy(i0 + 2, 0).start()

        copy(i0 + 1, 1).wait()
        acc = jax.lax.dot_general(
            wfull_ref[...], xbuf_ref[1],
            dimension_numbers=dn, preferred_element_type=jnp.float32)
        _routing_tail(acc, bias, scale, w_out_ref, i_out_ref,
                      (i0 + 1) * TILE_T)

        @pl.when(i0 + 3 < NT)
        def _pf1():
            copy(i0 + 3, 1).start()

        return carry

    lax.fori_loop(0, NT // 2, step, 0)


@jax.jit
def kernel(x, W_gate, W_up, extra_scale, extra_bias):
    bias2d = extra_bias.reshape(E, 1)
    scale2d = extra_scale.reshape(E, 1)
    weights, indices = pl.pallas_call(
        _router_kernel,
        in_specs=[
            pl.BlockSpec(memory_space=pl.ANY),
            pl.BlockSpec(memory_space=pltpu.VMEM),
            pl.BlockSpec(memory_space=pltpu.VMEM),
            pl.BlockSpec(memory_space=pltpu.VMEM),
            pl.BlockSpec(memory_space=pltpu.VMEM),
        ],
        out_specs=[
            pl.BlockSpec(memory_space=pltpu.VMEM),
            pl.BlockSpec(memory_space=pltpu.VMEM),
        ],
        out_shape=[
            jax.ShapeDtypeStruct((TOPK, T), jnp.float32),
            jax.ShapeDtypeStruct((TOPK, T), jnp.int32),
        ],
        scratch_shapes=[
            pltpu.VMEM((2, TILE_T, D), jnp.float32),
            pltpu.VMEM((D, 2 * E), jnp.float32),
            pltpu.SemaphoreType.DMA((2,)),
        ],
    )(x, W_gate, W_up, bias2d, scale2d)
    return weights.T, indices.T
